# baseline (speedup 1.0000x reference)
import functools
import math

import jax
import jax.numpy as jnp
from jax.experimental import pallas as pl
from jax.experimental.pallas import tpu as pltpu
from jax.experimental.pallas import tpu_sc as plsc

_N = 20000
_NPAD = 20480
_NCHUNK = 20
_K = 1000
_NMS_THRESH = 0.7
_IMG_H = 800.0
_IMG_W = 800.0
_SCALE_CLAMP = math.log(1000.0 / 16.0)
_NEG = -1e9
_NINF = float("-inf")


_NW = 32
_PERW = _NPAD // _NW
_SCMESH = plsc.VectorSubcoreMesh(core_axis_name="c", subcore_axis_name="s")


@functools.partial(
    pl.kernel,
    out_type=[jax.ShapeDtypeStruct((_NPAD,), jnp.float32)] * 5,
    mesh=_SCMESH,
    scratch_types=[pltpu.VMEM((_PERW,), jnp.float32)] * 14,
)
def _decode_sc(x1h, y1h, x2h, y2h, dxh, dyh, dwh, dhh, sch,
               ox1, oy1, ox2, oy2, oms,
               vx1, vy1, vx2, vy2, vdx, vdy, vdw, vdh, vsc,
               bx1, by1, bx2, by2, bms):
    wid = jax.lax.axis_index("s") * 2 + jax.lax.axis_index("c")
    base = wid * _PERW
    for src, dst in ((x1h, vx1), (y1h, vy1), (x2h, vx2), (y2h, vy2),
                     (dxh, vdx), (dyh, vdy), (dwh, vdw), (dhh, vdh),
                     (sch, vsc)):
        pltpu.sync_copy(src.at[pl.ds(base, _PERW)], dst)
    for j in range(_PERW // 16):
        sl = pl.ds(j * 16, 16)
        ax1 = vx1[sl]
        ay1 = vy1[sl]
        w = vx2[sl] - ax1
        h = vy2[sl] - ay1
        cx = ax1 + 0.5 * w
        cy = ay1 + 0.5 * h
        pcx = vdx[sl] * w + cx
        pcy = vdy[sl] * h + cy
        pw = jnp.exp(jnp.minimum(vdw[sl], _SCALE_CLAMP)) * w
        ph = jnp.exp(jnp.minimum(vdh[sl], _SCALE_CLAMP)) * h
        x1 = jnp.clip(pcx - 0.5 * pw, 0.0, _IMG_W)
        y1 = jnp.clip(pcy - 0.5 * ph, 0.0, _IMG_H)
        x2 = jnp.clip(pcx + 0.5 * pw, 0.0, _IMG_W)
        y2 = jnp.clip(pcy + 0.5 * ph, 0.0, _IMG_H)
        valid = ((x2 - x1) > 0.0) & ((y2 - y1) > 0.0)
        bx1[sl] = x1
        by1[sl] = y1
        bx2[sl] = x2
        by2[sl] = y2
        bms[sl] = jnp.where(valid, vsc[sl], _NEG)
    for src, dst in ((bx1, ox1), (by1, oy1), (bx2, ox2), (by2, oy2),
                     (bms, oms)):
        pltpu.sync_copy(src, dst.at[pl.ds(base, _PERW)])


def _flat_iota():
    s = jax.lax.broadcasted_iota(jnp.int32, (8, 128), 0)
    l = jax.lax.broadcasted_iota(jnp.int32, (8, 128), 1)
    return s * 128 + l


def _rpn_body(px1, py1, px2, py2, ms_in,
              ox1, oy1, ox2, oy2, osc,
              pms, smx1, smy1, smx2, smy2, sms):
    pms[...] = ms_in[...]

    iota2 = _flat_iota()
    zeros = jnp.zeros((8, 128), jnp.float32)

    def initcm_body(c, cm):
        return jnp.where(iota2 == c, jnp.max(pms[pl.ds(c, 1)]), cm)

    cm0 = jax.lax.fori_loop(0, _NCHUNK, initcm_body,
                            jnp.full((8, 128), _NINF))

    def extract_body(i, carry):
        cm, tx1, ty1, tx2, ty2, ts = carry
        m = jnp.max(cm)
        c = jnp.min(jnp.where(cm == m, iota2, jnp.int32(10 ** 9)))
        chunk = pms[pl.ds(c, 1)][0]
        li = jnp.min(jnp.where(chunk == m, iota2, jnp.int32(10 ** 9)))
        oh = iota2 == li
        x1i = jnp.sum(jnp.where(oh, px1[pl.ds(c, 1)][0], zeros))
        y1i = jnp.sum(jnp.where(oh, py1[pl.ds(c, 1)][0], zeros))
        x2i = jnp.sum(jnp.where(oh, px2[pl.ds(c, 1)][0], zeros))
        y2i = jnp.sum(jnp.where(oh, py2[pl.ds(c, 1)][0], zeros))
        newchunk = jnp.where(oh, _NINF, chunk)
        pms[pl.ds(c, 1)] = newchunk[None]
        cm = jnp.where(iota2 == c, jnp.max(newchunk), cm)
        smx1[i] = x1i
        smy1[i] = y1i
        smx2[i] = x2i
        smy2[i] = y2i
        sms[i] = m
        slot = iota2 == i
        tx1 = jnp.where(slot, x1i, tx1)
        ty1 = jnp.where(slot, y1i, ty1)
        tx2 = jnp.where(slot, x2i, tx2)
        ty2 = jnp.where(slot, y2i, ty2)
        ts = jnp.where(slot, m, ts)
        return cm, tx1, ty1, tx2, ty2, ts

    init = (cm0, zeros, zeros, zeros, zeros, jnp.full((8, 128), _NEG))
    _, tx1, ty1, tx2, ty2, ts = jax.lax.fori_loop(0, _K, extract_body, init)

    ta = (tx2 - tx1) * (ty2 - ty1)

    def nms_body(i, carry):
        keep, kcnt, rx1, ry1, rx2, ry2, rs = carry
        ohi = iota2 == i
        x1i = smx1[i]
        y1i = smy1[i]
        x2i = smx2[i]
        y2i = smy2[i]
        si = sms[i]
        ai = (x2i - x1i) * (y2i - y1i)
        alive = jnp.sum(jnp.where(ohi, keep, zeros)) > 0.5
        iw = jnp.maximum(jnp.minimum(x2i, tx2) - jnp.maximum(x1i, tx1), 0.0)
        ih = jnp.maximum(jnp.minimum(y2i, ty2) - jnp.maximum(y1i, ty1), 0.0)
        inter = iw * ih
        union = ai + ta - inter
        iou = jnp.where(union > 0.0,
                        inter / jnp.maximum(union, 1e-9), 0.0)
        sup = (iou > _NMS_THRESH) & (iota2 > i) & alive
        keep = jnp.where(sup, 0.0, keep)
        g = alive & (si > _NEG * 0.5)
        oslot = (iota2 == kcnt) & g
        rx1 = jnp.where(oslot, x1i, rx1)
        ry1 = jnp.where(oslot, y1i, ry1)
        rx2 = jnp.where(oslot, x2i, rx2)
        ry2 = jnp.where(oslot, y2i, ry2)
        rs = jnp.where(oslot, si, rs)
        kcnt = kcnt + jnp.where(g, 1, 0).astype(jnp.int32)
        return keep, kcnt, rx1, ry1, rx2, ry2, rs

    init_c = (jnp.ones((8, 128), jnp.float32), jnp.int32(0),
              zeros, zeros, zeros, zeros, zeros)
    _, _, rx1, ry1, rx2, ry2, rs = jax.lax.fori_loop(0, _K, nms_body, init_c)

    ox1[...] = rx1
    oy1[...] = ry1
    ox2[...] = rx2
    oy2[...] = ry2
    osc[...] = rs


def kernel(anchors, deltas, scores):
    at = jnp.pad(anchors, ((0, _NPAD - _N), (0, 0))).T
    dt = jnp.pad(deltas, ((0, _NPAD - _N), (0, 0))).T
    sp = jnp.pad(scores, (0, _NPAD - _N))
    decoded = _decode_sc(at[0], at[1], at[2], at[3],
                         dt[0], dt[1], dt[2], dt[3], sp)
    planes = [d.reshape(_NCHUNK, 8, 128) for d in decoded]
    out = pl.pallas_call(
        _rpn_body,
        out_shape=[jax.ShapeDtypeStruct((8, 128), jnp.float32)] * 5,
        scratch_shapes=[pltpu.VMEM((_NCHUNK, 8, 128), jnp.float32)]
        + [pltpu.SMEM((_K,), jnp.float32)] * 5,
    )(*planes)
    cols = [o.reshape(_NPAD // _NCHUNK)[:_K] for o in out]
    return jnp.stack(cols, axis=-1)


# speedup vs baseline: 1.0033x; 1.0033x over previous
import functools
import math

import jax
import jax.numpy as jnp
from jax.experimental import pallas as pl
from jax.experimental.pallas import tpu as pltpu
from jax.experimental.pallas import tpu_sc as plsc

_N = 20000
_NPAD = 20480
_NCHUNK = 20
_K = 1000
_NMS_THRESH = 0.7
_IMG_H = 800.0
_IMG_W = 800.0
_SCALE_CLAMP = math.log(1000.0 / 16.0)
_NEG = -1e9
_NINF = float("-inf")


_NW = 32
_PERW = _NPAD // _NW
_SCMESH = plsc.VectorSubcoreMesh(core_axis_name="c", subcore_axis_name="s")


@functools.partial(
    pl.kernel,
    out_type=[jax.ShapeDtypeStruct((_NPAD,), jnp.float32)] * 5,
    mesh=_SCMESH,
    scratch_types=[pltpu.VMEM((_PERW,), jnp.float32)] * 14,
)
def _decode_sc(x1h, y1h, x2h, y2h, dxh, dyh, dwh, dhh, sch,
               ox1, oy1, ox2, oy2, oms,
               vx1, vy1, vx2, vy2, vdx, vdy, vdw, vdh, vsc,
               bx1, by1, bx2, by2, bms):
    wid = jax.lax.axis_index("s") * 2 + jax.lax.axis_index("c")
    base = wid * _PERW
    for src, dst in ((x1h, vx1), (y1h, vy1), (x2h, vx2), (y2h, vy2),
                     (dxh, vdx), (dyh, vdy), (dwh, vdw), (dhh, vdh),
                     (sch, vsc)):
        pltpu.sync_copy(src.at[pl.ds(base, _PERW)], dst)
    for j in range(_PERW // 16):
        sl = pl.ds(j * 16, 16)
        ax1 = vx1[sl]
        ay1 = vy1[sl]
        w = vx2[sl] - ax1
        h = vy2[sl] - ay1
        cx = ax1 + 0.5 * w
        cy = ay1 + 0.5 * h
        pcx = vdx[sl] * w + cx
        pcy = vdy[sl] * h + cy
        pw = jnp.exp(jnp.minimum(vdw[sl], _SCALE_CLAMP)) * w
        ph = jnp.exp(jnp.minimum(vdh[sl], _SCALE_CLAMP)) * h
        x1 = jnp.clip(pcx - 0.5 * pw, 0.0, _IMG_W)
        y1 = jnp.clip(pcy - 0.5 * ph, 0.0, _IMG_H)
        x2 = jnp.clip(pcx + 0.5 * pw, 0.0, _IMG_W)
        y2 = jnp.clip(pcy + 0.5 * ph, 0.0, _IMG_H)
        valid = ((x2 - x1) > 0.0) & ((y2 - y1) > 0.0)
        bx1[sl] = x1
        by1[sl] = y1
        bx2[sl] = x2
        by2[sl] = y2
        bms[sl] = jnp.where(valid, vsc[sl], _NEG)
    for src, dst in ((bx1, ox1), (by1, oy1), (bx2, ox2), (by2, oy2),
                     (bms, oms)):
        pltpu.sync_copy(src, dst.at[pl.ds(base, _PERW)])


def _flat_iota():
    s = jax.lax.broadcasted_iota(jnp.int32, (8, 128), 0)
    l = jax.lax.broadcasted_iota(jnp.int32, (8, 128), 1)
    return s * 128 + l


def _rpn_body(px1, py1, px2, py2, ms_in,
              ox1, oy1, ox2, oy2, osc,
              pms, smx1, smy1, smx2, smy2, sms):
    pms[...] = ms_in[...]

    iota2 = _flat_iota()
    zeros = jnp.zeros((8, 128), jnp.float32)

    def initcm_body(c, cm):
        return jnp.where(iota2 == c, jnp.max(pms[pl.ds(c, 1)]), cm)

    cm0 = jax.lax.fori_loop(0, _NCHUNK, initcm_body,
                            jnp.full((8, 128), _NINF))

    def extract_body(i, carry):
        cm, tx1, ty1, tx2, ty2, ts = carry
        m = jnp.max(cm)
        c = jnp.min(jnp.where(cm == m, iota2, jnp.int32(10 ** 9)))
        chunk = pms[pl.ds(c, 1)][0]
        li = jnp.min(jnp.where(chunk == m, iota2, jnp.int32(10 ** 9)))
        oh = iota2 == li
        x1i = jnp.sum(jnp.where(oh, px1[pl.ds(c, 1)][0], zeros))
        y1i = jnp.sum(jnp.where(oh, py1[pl.ds(c, 1)][0], zeros))
        x2i = jnp.sum(jnp.where(oh, px2[pl.ds(c, 1)][0], zeros))
        y2i = jnp.sum(jnp.where(oh, py2[pl.ds(c, 1)][0], zeros))
        newchunk = jnp.where(oh, _NINF, chunk)
        pms[pl.ds(c, 1)] = newchunk[None]
        cm = jnp.where(iota2 == c, jnp.max(newchunk), cm)
        smx1[i] = x1i
        smy1[i] = y1i
        smx2[i] = x2i
        smy2[i] = y2i
        sms[i] = m
        slot = iota2 == i
        tx1 = jnp.where(slot, x1i, tx1)
        ty1 = jnp.where(slot, y1i, ty1)
        tx2 = jnp.where(slot, x2i, tx2)
        ty2 = jnp.where(slot, y2i, ty2)
        ts = jnp.where(slot, m, ts)
        return cm, tx1, ty1, tx2, ty2, ts

    init = (cm0, zeros, zeros, zeros, zeros, jnp.full((8, 128), _NEG))
    _, tx1, ty1, tx2, ty2, ts = jax.lax.fori_loop(0, _K, extract_body, init,
                                                  unroll=4)

    ta = (tx2 - tx1) * (ty2 - ty1)

    def nms_body(i, carry):
        keep, kcnt, rx1, ry1, rx2, ry2, rs = carry
        ohi = iota2 == i
        x1i = smx1[i]
        y1i = smy1[i]
        x2i = smx2[i]
        y2i = smy2[i]
        si = sms[i]
        ai = (x2i - x1i) * (y2i - y1i)
        alive = jnp.sum(jnp.where(ohi, keep, zeros)) > 0.5
        iw = jnp.maximum(jnp.minimum(x2i, tx2) - jnp.maximum(x1i, tx1), 0.0)
        ih = jnp.maximum(jnp.minimum(y2i, ty2) - jnp.maximum(y1i, ty1), 0.0)
        inter = iw * ih
        union = ai + ta - inter
        iou = jnp.where(union > 0.0,
                        inter / jnp.maximum(union, 1e-9), 0.0)
        sup = (iou > _NMS_THRESH) & (iota2 > i) & alive
        keep = jnp.where(sup, 0.0, keep)
        g = alive & (si > _NEG * 0.5)
        oslot = (iota2 == kcnt) & g
        rx1 = jnp.where(oslot, x1i, rx1)
        ry1 = jnp.where(oslot, y1i, ry1)
        rx2 = jnp.where(oslot, x2i, rx2)
        ry2 = jnp.where(oslot, y2i, ry2)
        rs = jnp.where(oslot, si, rs)
        kcnt = kcnt + jnp.where(g, 1, 0).astype(jnp.int32)
        return keep, kcnt, rx1, ry1, rx2, ry2, rs

    init_c = (jnp.ones((8, 128), jnp.float32), jnp.int32(0),
              zeros, zeros, zeros, zeros, zeros)
    _, _, rx1, ry1, rx2, ry2, rs = jax.lax.fori_loop(0, _K, nms_body, init_c,
                                                     unroll=4)

    ox1[...] = rx1
    oy1[...] = ry1
    ox2[...] = rx2
    oy2[...] = ry2
    osc[...] = rs


def kernel(anchors, deltas, scores):
    at = jnp.pad(anchors, ((0, _NPAD - _N), (0, 0))).T
    dt = jnp.pad(deltas, ((0, _NPAD - _N), (0, 0))).T
    sp = jnp.pad(scores, (0, _NPAD - _N))
    decoded = _decode_sc(at[0], at[1], at[2], at[3],
                         dt[0], dt[1], dt[2], dt[3], sp)
    planes = [d.reshape(_NCHUNK, 8, 128) for d in decoded]
    out = pl.pallas_call(
        _rpn_body,
        out_shape=[jax.ShapeDtypeStruct((8, 128), jnp.float32)] * 5,
        scratch_shapes=[pltpu.VMEM((_NCHUNK, 8, 128), jnp.float32)]
        + [pltpu.SMEM((_K,), jnp.float32)] * 5,
    )(*planes)
    cols = [o.reshape(_NPAD // _NCHUNK)[:_K] for o in out]
    return jnp.stack(cols, axis=-1)


# speedup vs baseline: 4.5523x; 4.5375x over previous
import functools
import math

import jax
import jax.numpy as jnp
from jax.experimental import pallas as pl
from jax.experimental.pallas import tpu as pltpu
from jax.experimental.pallas import tpu_sc as plsc

_N = 20000
_NPAD = 20480
_NCHUNK = 20
_K = 1000
_NMS_THRESH = 0.7
_IMG_H = 800.0
_IMG_W = 800.0
_SCALE_CLAMP = math.log(1000.0 / 16.0)
_NEG = -1e9
_NINF = float("-inf")
_BIGIDX = 1 << 28

_NW = 32
_PERW = _NPAD // _NW
_GPERW = 1024 // _NW
_SCMESH = plsc.VectorSubcoreMesh(core_axis_name="c", subcore_axis_name="s")


@functools.partial(
    pl.kernel,
    out_type=[jax.ShapeDtypeStruct((_NPAD,), jnp.float32)] * 5,
    mesh=_SCMESH,
    scratch_types=[pltpu.VMEM((_PERW,), jnp.float32)] * 14,
)
def _decode_sc(x1h, y1h, x2h, y2h, dxh, dyh, dwh, dhh, sch,
               ox1, oy1, ox2, oy2, oms,
               vx1, vy1, vx2, vy2, vdx, vdy, vdw, vdh, vsc,
               bx1, by1, bx2, by2, bms):
    wid = jax.lax.axis_index("s") * 2 + jax.lax.axis_index("c")
    base = wid * _PERW
    for src, dst in ((x1h, vx1), (y1h, vy1), (x2h, vx2), (y2h, vy2),
                     (dxh, vdx), (dyh, vdy), (dwh, vdw), (dhh, vdh),
                     (sch, vsc)):
        pltpu.sync_copy(src.at[pl.ds(base, _PERW)], dst)
    for j in range(_PERW // 16):
        sl = pl.ds(j * 16, 16)
        ax1 = vx1[sl]
        ay1 = vy1[sl]
        w = vx2[sl] - ax1
        h = vy2[sl] - ay1
        cx = ax1 + 0.5 * w
        cy = ay1 + 0.5 * h
        pcx = vdx[sl] * w + cx
        pcy = vdy[sl] * h + cy
        pw = jnp.exp(jnp.minimum(vdw[sl], _SCALE_CLAMP)) * w
        ph = jnp.exp(jnp.minimum(vdh[sl], _SCALE_CLAMP)) * h
        x1 = jnp.clip(pcx - 0.5 * pw, 0.0, _IMG_W)
        y1 = jnp.clip(pcy - 0.5 * ph, 0.0, _IMG_H)
        x2 = jnp.clip(pcx + 0.5 * pw, 0.0, _IMG_W)
        y2 = jnp.clip(pcy + 0.5 * ph, 0.0, _IMG_H)
        valid = ((x2 - x1) > 0.0) & ((y2 - y1) > 0.0)
        bx1[sl] = x1
        by1[sl] = y1
        bx2[sl] = x2
        by2[sl] = y2
        bms[sl] = jnp.where(valid, vsc[sl], _NEG)
    for src, dst in ((bx1, ox1), (by1, oy1), (bx2, ox2), (by2, oy2),
                     (bms, oms)):
        pltpu.sync_copy(src, dst.at[pl.ds(base, _PERW)])


@functools.partial(
    pl.kernel,
    out_type=[jax.ShapeDtypeStruct((1024,), jnp.float32)] * 4,
    mesh=_SCMESH,
    scratch_types=[pltpu.VMEM((_GPERW,), jnp.int32)]
    + [pltpu.VMEM((_GPERW,), jnp.float32)] * 4
    + [pltpu.SemaphoreType.DMA],
)
def _gather_sc(idxh, x1h, y1h, x2h, y2h, ox1, oy1, ox2, oy2,
               vidx, g1, g2, g3, g4, sem):
    wid = jax.lax.axis_index("s") * 2 + jax.lax.axis_index("c")
    base = wid * _GPERW
    pltpu.sync_copy(idxh.at[pl.ds(base, _GPERW)], vidx)
    for src, buf, dst in ((x1h, g1, ox1), (y1h, g2, oy1),
                          (x2h, g3, ox2), (y2h, g4, oy2)):
        pltpu.async_copy(src.at[vidx], buf, sem).wait()
        pltpu.sync_copy(buf, dst.at[pl.ds(base, _GPERW)])


def _flat_iota():
    s = jax.lax.broadcasted_iota(jnp.int32, (8, 128), 0)
    l = jax.lax.broadcasted_iota(jnp.int32, (8, 128), 1)
    return s * 128 + l


def _cmpx(s, ix, ps, pix, take_better):
    better = (s > ps) | ((s == ps) & (ix < pix))
    keep = better == take_better
    return jnp.where(keep, s, ps), jnp.where(keep, ix, pix)


def _substep_sub(s, ix, ds, dirb, iota2):
    mybit = ((iota2 & (ds * 128)) != 0)[None]
    ps = jnp.where(mybit, pltpu.roll(s, ds, axis=1),
                   pltpu.roll(s, 8 - ds, axis=1))
    pix = jnp.where(mybit, pltpu.roll(ix, ds, axis=1),
                    pltpu.roll(ix, 8 - ds, axis=1))
    return _cmpx(s, ix, ps, pix, mybit == dirb)


def _substep_lane(s, ix, d, dirb, iota2):
    mybit = ((iota2 & d) != 0)[None]
    ps = jnp.where(mybit, pltpu.roll(s, d, axis=2),
                   pltpu.roll(s, 128 - d, axis=2))
    pix = jnp.where(mybit, pltpu.roll(ix, d, axis=2),
                    pltpu.roll(ix, 128 - d, axis=2))
    return _cmpx(s, ix, ps, pix, mybit == dirb)


def _sort_body(msr, osc, oidx):
    iota2 = _flat_iota()
    s = msr[...]
    c3 = jax.lax.broadcasted_iota(jnp.int32, (_NCHUNK, 8, 128), 0)
    ix = c3 * 1024 + iota2[None]
    cdir = c3 >= 16

    for k in range(1, 11):
        dirb = (((iota2 >> k) & 1) != 0)[None] ^ cdir
        d = 1 << (k - 1)
        while d >= 128:
            s, ix = _substep_sub(s, ix, d // 128, dirb, iota2)
            d //= 2
        start = min(1 << (k - 1), 64)

        def lane_body(j, carry, start=start, dirb=dirb):
            ss, ii = carry
            return _substep_lane(ss, ii, start >> j, dirb, iota2)

        s, ix = jax.lax.fori_loop(0, start.bit_length(), lane_body, (s, ix))

    s = jnp.concatenate(
        [s, jnp.full((_NW - _NCHUNK, 8, 128), _NINF, jnp.float32)], 0)
    ix = jnp.concatenate(
        [ix, jnp.full((_NW - _NCHUNK, 8, 128), _BIGIDX, jnp.int32)], 0)
    m = _NW // 2
    while m >= 1:
        sa, sb = s[:m], s[m:2 * m]
        ia, ib = ix[:m], ix[m:2 * m]
        better = (sa > sb) | ((sa == sb) & (ia < ib))
        s = jnp.where(better, sa, sb)
        ix = jnp.where(better, ia, ib)
        if m > 1:
            dirb = jax.lax.broadcasted_iota(
                jnp.int32, (m, 8, 128), 0) >= (m // 2)
        else:
            dirb = jnp.zeros((1, 8, 128), jnp.bool_)
        for ds in (4, 2, 1):
            s, ix = _substep_sub(s, ix, ds, dirb, iota2)

        def lane_body(j, carry, dirb=dirb):
            ss, ii = carry
            return _substep_lane(ss, ii, 64 >> j, dirb, iota2)

        s, ix = jax.lax.fori_loop(0, 7, lane_body, (s, ix))
        m //= 2

    osc[...] = s[0]
    oidx[...] = ix[0]


def _nms_body(tx1r, ty1r, tx2r, ty2r,
              sx1, sy1, sx2, sy2, ssc,
              ox1, oy1, ox2, oy2, osc):
    iota2 = _flat_iota()
    zeros = jnp.zeros((8, 128), jnp.float32)
    tx1 = tx1r[...]
    ty1 = ty1r[...]
    tx2 = tx2r[...]
    ty2 = ty2r[...]
    ta = (tx2 - tx1) * (ty2 - ty1)

    def nms_body(i, carry):
        keep, kcnt, rx1, ry1, rx2, ry2, rs = carry
        ohi = iota2 == i
        x1i = sx1[i]
        y1i = sy1[i]
        x2i = sx2[i]
        y2i = sy2[i]
        si = ssc[i]
        ai = (x2i - x1i) * (y2i - y1i)
        alive = jnp.sum(jnp.where(ohi, keep, zeros)) > 0.5
        iw = jnp.maximum(jnp.minimum(x2i, tx2) - jnp.maximum(x1i, tx1), 0.0)
        ih = jnp.maximum(jnp.minimum(y2i, ty2) - jnp.maximum(y1i, ty1), 0.0)
        inter = iw * ih
        union = ai + ta - inter
        iou = jnp.where(union > 0.0,
                        inter / jnp.maximum(union, 1e-9), 0.0)
        sup = (iou > _NMS_THRESH) & (iota2 > i) & alive
        keep = jnp.where(sup, 0.0, keep)
        g = alive & (si > _NEG * 0.5)
        oslot = (iota2 == kcnt) & g
        rx1 = jnp.where(oslot, x1i, rx1)
        ry1 = jnp.where(oslot, y1i, ry1)
        rx2 = jnp.where(oslot, x2i, rx2)
        ry2 = jnp.where(oslot, y2i, ry2)
        rs = jnp.where(oslot, si, rs)
        kcnt = kcnt + jnp.where(g, 1, 0).astype(jnp.int32)
        return keep, kcnt, rx1, ry1, rx2, ry2, rs

    init_c = (jnp.ones((8, 128), jnp.float32), jnp.int32(0),
              zeros, zeros, zeros, zeros, zeros)
    _, _, rx1, ry1, rx2, ry2, rs = jax.lax.fori_loop(0, _K, nms_body, init_c)

    ox1[...] = rx1
    oy1[...] = ry1
    ox2[...] = rx2
    oy2[...] = ry2
    osc[...] = rs


def kernel(anchors, deltas, scores):
    at = jnp.pad(anchors, ((0, _NPAD - _N), (0, 0))).T
    dt = jnp.pad(deltas, ((0, _NPAD - _N), (0, 0))).T
    sp = jnp.pad(scores, (0, _NPAD - _N))
    dx1, dy1, dx2, dy2, dms = _decode_sc(at[0], at[1], at[2], at[3],
                                         dt[0], dt[1], dt[2], dt[3], sp)
    ssort, isort = pl.pallas_call(
        _sort_body,
        out_shape=[jax.ShapeDtypeStruct((8, 128), jnp.float32),
                   jax.ShapeDtypeStruct((8, 128), jnp.int32)],
    )(dms.reshape(_NCHUNK, 8, 128))
    gx1, gy1, gx2, gy2 = _gather_sc(isort.reshape(1024), dx1, dy1, dx2, dy2)
    out = pl.pallas_call(
        _nms_body,
        out_shape=[jax.ShapeDtypeStruct((8, 128), jnp.float32)] * 5,
        in_specs=[pl.BlockSpec(memory_space=pltpu.VMEM)] * 4
        + [pl.BlockSpec(memory_space=pltpu.SMEM)] * 5,
    )(gx1.reshape(8, 128), gy1.reshape(8, 128),
      gx2.reshape(8, 128), gy2.reshape(8, 128),
      gx1, gy1, gx2, gy2, ssort.reshape(1024))
    cols = [o.reshape(1024)[:_K] for o in out]
    return jnp.stack(cols, axis=-1)


# speedup vs baseline: 4.6231x; 1.0156x over previous
import functools
import math

import jax
import jax.numpy as jnp
from jax.experimental import pallas as pl
from jax.experimental.pallas import tpu as pltpu
from jax.experimental.pallas import tpu_sc as plsc

_N = 20000
_NPAD = 20480
_NCHUNK = 20
_K = 1000
_NMS_THRESH = 0.7
_IMG_H = 800.0
_IMG_W = 800.0
_SCALE_CLAMP = math.log(1000.0 / 16.0)
_NEG = -1e9
_NINF = float("-inf")
_BIGIDX = 1 << 28

_NW = 32
_PERW = _NPAD // _NW
_GPERW = 1024 // _NW
_SCMESH = plsc.VectorSubcoreMesh(core_axis_name="c", subcore_axis_name="s")


@functools.partial(
    pl.kernel,
    out_type=[jax.ShapeDtypeStruct((_NPAD,), jnp.float32)] * 5,
    mesh=_SCMESH,
    scratch_types=[pltpu.VMEM((_PERW,), jnp.float32)] * 14,
)
def _decode_sc(x1h, y1h, x2h, y2h, dxh, dyh, dwh, dhh, sch,
               ox1, oy1, ox2, oy2, oms,
               vx1, vy1, vx2, vy2, vdx, vdy, vdw, vdh, vsc,
               bx1, by1, bx2, by2, bms):
    wid = jax.lax.axis_index("s") * 2 + jax.lax.axis_index("c")
    base = wid * _PERW
    for src, dst in ((x1h, vx1), (y1h, vy1), (x2h, vx2), (y2h, vy2),
                     (dxh, vdx), (dyh, vdy), (dwh, vdw), (dhh, vdh),
                     (sch, vsc)):
        pltpu.sync_copy(src.at[pl.ds(base, _PERW)], dst)
    for j in range(_PERW // 16):
        sl = pl.ds(j * 16, 16)
        ax1 = vx1[sl]
        ay1 = vy1[sl]
        w = vx2[sl] - ax1
        h = vy2[sl] - ay1
        cx = ax1 + 0.5 * w
        cy = ay1 + 0.5 * h
        pcx = vdx[sl] * w + cx
        pcy = vdy[sl] * h + cy
        pw = jnp.exp(jnp.minimum(vdw[sl], _SCALE_CLAMP)) * w
        ph = jnp.exp(jnp.minimum(vdh[sl], _SCALE_CLAMP)) * h
        x1 = jnp.clip(pcx - 0.5 * pw, 0.0, _IMG_W)
        y1 = jnp.clip(pcy - 0.5 * ph, 0.0, _IMG_H)
        x2 = jnp.clip(pcx + 0.5 * pw, 0.0, _IMG_W)
        y2 = jnp.clip(pcy + 0.5 * ph, 0.0, _IMG_H)
        valid = ((x2 - x1) > 0.0) & ((y2 - y1) > 0.0)
        bx1[sl] = x1
        by1[sl] = y1
        bx2[sl] = x2
        by2[sl] = y2
        bms[sl] = jnp.where(valid, vsc[sl], _NEG)
    for src, dst in ((bx1, ox1), (by1, oy1), (bx2, ox2), (by2, oy2),
                     (bms, oms)):
        pltpu.sync_copy(src, dst.at[pl.ds(base, _PERW)])


@functools.partial(
    pl.kernel,
    out_type=[jax.ShapeDtypeStruct((1024,), jnp.float32)] * 4,
    mesh=_SCMESH,
    scratch_types=[pltpu.VMEM((_GPERW,), jnp.int32)]
    + [pltpu.VMEM((_GPERW,), jnp.float32)] * 4
    + [pltpu.SemaphoreType.DMA],
)
def _gather_sc(idxh, x1h, y1h, x2h, y2h, ox1, oy1, ox2, oy2,
               vidx, g1, g2, g3, g4, sem):
    wid = jax.lax.axis_index("s") * 2 + jax.lax.axis_index("c")
    base = wid * _GPERW
    pltpu.sync_copy(idxh.at[pl.ds(base, _GPERW)], vidx)
    for src, buf, dst in ((x1h, g1, ox1), (y1h, g2, oy1),
                          (x2h, g3, ox2), (y2h, g4, oy2)):
        pltpu.async_copy(src.at[vidx], buf, sem).wait()
        pltpu.sync_copy(buf, dst.at[pl.ds(base, _GPERW)])


def _flat_iota():
    s = jax.lax.broadcasted_iota(jnp.int32, (8, 128), 0)
    l = jax.lax.broadcasted_iota(jnp.int32, (8, 128), 1)
    return s * 128 + l


def _cmpx(s, ix, ps, pix, take_better):
    better = (s > ps) | ((s == ps) & (ix < pix))
    keep = better == take_better
    return jnp.where(keep, s, ps), jnp.where(keep, ix, pix)


def _substep_sub(s, ix, ds, dirb, iota2):
    mybit = ((iota2 & (ds * 128)) != 0)[None]
    ps = jnp.where(mybit, pltpu.roll(s, ds, axis=1),
                   pltpu.roll(s, 8 - ds, axis=1))
    pix = jnp.where(mybit, pltpu.roll(ix, ds, axis=1),
                    pltpu.roll(ix, 8 - ds, axis=1))
    return _cmpx(s, ix, ps, pix, mybit == dirb)


def _substep_lane(s, ix, d, dirb, iota2):
    mybit = ((iota2 & d) != 0)[None]
    ps = jnp.where(mybit, pltpu.roll(s, d, axis=2),
                   pltpu.roll(s, 128 - d, axis=2))
    pix = jnp.where(mybit, pltpu.roll(ix, d, axis=2),
                    pltpu.roll(ix, 128 - d, axis=2))
    return _cmpx(s, ix, ps, pix, mybit == dirb)


def _sort_body(msr, osc, oidx):
    iota2 = _flat_iota()
    s = msr[...]
    c3 = jax.lax.broadcasted_iota(jnp.int32, (_NCHUNK, 8, 128), 0)
    ix = c3 * 1024 + iota2[None]
    cdir = c3 >= 16

    for k in range(1, 11):
        dirb = (((iota2 >> k) & 1) != 0)[None] ^ cdir
        d = 1 << (k - 1)
        while d >= 128:
            s, ix = _substep_sub(s, ix, d // 128, dirb, iota2)
            d //= 2
        start = min(1 << (k - 1), 64)

        def lane_body(j, carry, start=start, dirb=dirb):
            ss, ii = carry
            return _substep_lane(ss, ii, start >> j, dirb, iota2)

        s, ix = jax.lax.fori_loop(0, start.bit_length(), lane_body, (s, ix))

    s = jnp.concatenate(
        [s, jnp.full((_NW - _NCHUNK, 8, 128), _NINF, jnp.float32)], 0)
    ix = jnp.concatenate(
        [ix, jnp.full((_NW - _NCHUNK, 8, 128), _BIGIDX, jnp.int32)], 0)
    m = _NW // 2
    while m >= 1:
        sa, sb = s[:m], s[m:2 * m]
        ia, ib = ix[:m], ix[m:2 * m]
        better = (sa > sb) | ((sa == sb) & (ia < ib))
        s = jnp.where(better, sa, sb)
        ix = jnp.where(better, ia, ib)
        if m > 1:
            dirb = jax.lax.broadcasted_iota(
                jnp.int32, (m, 8, 128), 0) >= (m // 2)
        else:
            dirb = jnp.zeros((1, 8, 128), jnp.bool_)
        for ds in (4, 2, 1):
            s, ix = _substep_sub(s, ix, ds, dirb, iota2)

        def lane_body(j, carry, dirb=dirb):
            ss, ii = carry
            return _substep_lane(ss, ii, 64 >> j, dirb, iota2)

        s, ix = jax.lax.fori_loop(0, 7, lane_body, (s, ix))
        m //= 2

    osc[...] = s[0]
    oidx[...] = ix[0]


def _nms_body(tx1r, ty1r, tx2r, ty2r,
              sx1, sy1, sx2, sy2, ssc,
              ox1, oy1, ox2, oy2, osc):
    iota2 = _flat_iota()
    zeros = jnp.zeros((8, 128), jnp.float32)
    tx1 = tx1r[...]
    ty1 = ty1r[...]
    tx2 = tx2r[...]
    ty2 = ty2r[...]
    ta = (tx2 - tx1) * (ty2 - ty1)

    def nms_body(i, carry):
        keep, kcnt, rx1, ry1, rx2, ry2, rs = carry
        ohi = iota2 == i
        x1i = sx1[i]
        y1i = sy1[i]
        x2i = sx2[i]
        y2i = sy2[i]
        si = ssc[i]
        ai = (x2i - x1i) * (y2i - y1i)
        alive = jnp.max(jnp.where(ohi, keep, zeros),
                        axis=(0, 1), keepdims=True) > 0.5
        iw = jnp.maximum(jnp.minimum(x2i, tx2) - jnp.maximum(x1i, tx1), 0.0)
        ih = jnp.maximum(jnp.minimum(y2i, ty2) - jnp.maximum(y1i, ty1), 0.0)
        inter = iw * ih
        union = ai + ta - inter
        iou = jnp.where(union > 0.0,
                        inter / jnp.maximum(union, 1e-9), 0.0)
        sup = (iou > _NMS_THRESH) & (iota2 > i) & alive
        keep = jnp.where(sup, 0.0, keep)
        g = alive & (si > _NEG * 0.5)
        oslot = (iota2 == kcnt) & g
        rx1 = jnp.where(oslot, x1i, rx1)
        ry1 = jnp.where(oslot, y1i, ry1)
        rx2 = jnp.where(oslot, x2i, rx2)
        ry2 = jnp.where(oslot, y2i, ry2)
        rs = jnp.where(oslot, si, rs)
        kcnt = kcnt + jnp.where(g, 1, 0).astype(jnp.int32)
        return keep, kcnt, rx1, ry1, rx2, ry2, rs

    init_c = (jnp.ones((8, 128), jnp.float32),
              jnp.zeros((8, 128), jnp.int32),
              zeros, zeros, zeros, zeros, zeros)
    _, _, rx1, ry1, rx2, ry2, rs = jax.lax.fori_loop(0, _K, nms_body, init_c)

    ox1[...] = rx1
    oy1[...] = ry1
    ox2[...] = rx2
    oy2[...] = ry2
    osc[...] = rs


def kernel(anchors, deltas, scores):
    at = jnp.pad(anchors, ((0, _NPAD - _N), (0, 0))).T
    dt = jnp.pad(deltas, ((0, _NPAD - _N), (0, 0))).T
    sp = jnp.pad(scores, (0, _NPAD - _N))
    dx1, dy1, dx2, dy2, dms = _decode_sc(at[0], at[1], at[2], at[3],
                                         dt[0], dt[1], dt[2], dt[3], sp)
    ssort, isort = pl.pallas_call(
        _sort_body,
        out_shape=[jax.ShapeDtypeStruct((8, 128), jnp.float32),
                   jax.ShapeDtypeStruct((8, 128), jnp.int32)],
    )(dms.reshape(_NCHUNK, 8, 128))
    gx1, gy1, gx2, gy2 = _gather_sc(isort.reshape(1024), dx1, dy1, dx2, dy2)
    out = pl.pallas_call(
        _nms_body,
        out_shape=[jax.ShapeDtypeStruct((8, 128), jnp.float32)] * 5,
        in_specs=[pl.BlockSpec(memory_space=pltpu.VMEM)] * 4
        + [pl.BlockSpec(memory_space=pltpu.SMEM)] * 5,
    )(gx1.reshape(8, 128), gy1.reshape(8, 128),
      gx2.reshape(8, 128), gy2.reshape(8, 128),
      gx1, gy1, gx2, gy2, ssort.reshape(1024))
    cols = [o.reshape(1024)[:_K] for o in out]
    return jnp.stack(cols, axis=-1)
